# baseline (speedup 1.0000x reference)
import functools

import jax
import jax.numpy as jnp
from jax import lax
from jax.experimental import pallas as pl
from jax.experimental.pallas import tpu as pltpu
from jax.experimental.pallas import tpu_sc as plsc

_NUM_CORES = 2
_NUM_SUBCORES = 16
_LANES = 16
_NUM_WORKERS = _NUM_CORES * _NUM_SUBCORES
_CHUNK = 40
_NBUF = 2


def _sc_gather(x_pad, table, B, TP):
    (N,) = x_pad.shape
    V, D = table.shape
    BW = B // _NUM_WORKERS
    RP = BW * TP
    CPB = TP // _CHUNK
    NCH = BW * CPB

    mesh = plsc.VectorSubcoreMesh(core_axis_name="c", subcore_axis_name="s")

    @functools.partial(
        pl.kernel,
        out_type=jax.ShapeDtypeStruct((B, TP, D), jnp.float32),
        mesh=mesh,
        compiler_params=pltpu.CompilerParams(use_tc_tiling_on_sc=True),
        scratch_types=[
            pltpu.VMEM((RP,), jnp.int32),
            pltpu.VMEM((_CHUNK, D), jnp.float32),
            pltpu.VMEM((_CHUNK, D), jnp.float32),
            pltpu.SemaphoreType.DMA,
            pltpu.SemaphoreType.DMA,
            pltpu.SemaphoreType.DMA,
            pltpu.SemaphoreType.DMA,
        ],
    )
    def k(x_hbm, tab_hbm, out_hbm, idx_v, lin0, lin1, g0, g1, s0, s1):
        wid = lax.axis_index("s") * _NUM_CORES + lax.axis_index("c")
        b0 = wid * BW
        lins = [lin0, lin1]
        gsems = [g0, g1]
        ssems = [s0, s1]

        pltpu.sync_copy(x_hbm.at[pl.ds(wid * RP, RP)], idx_v)

        def start_gather(c, slot):
            pltpu.async_copy(
                tab_hbm.at[idx_v.at[pl.ds(c * _CHUNK, _CHUNK)]],
                lins[slot], gsems[slot],
            )

        def wait_gather(slot):
            pltpu.make_async_copy(
                tab_hbm.at[idx_v.at[pl.ds(0, _CHUNK)]],
                lins[slot], gsems[slot],
            ).wait()

        def out_ref(c):
            bb = c // CPB
            piece = lax.rem(c, CPB)
            return out_hbm.at[b0 + bb, pl.ds(piece * _CHUNK, _CHUNK)]

        def start_store(c, slot):
            pltpu.async_copy(lins[slot], out_ref(c), ssems[slot])

        def wait_store(c, slot):
            pltpu.make_async_copy(lins[slot], out_ref(c), ssems[slot]).wait()

        for slot in range(_NBUF):
            start_gather(slot, slot)

        def pair_body(p, carry):
            for slot in range(_NBUF):
                c = p * _NBUF + slot
                wait_gather(slot)
                start_store(c, slot)
                wait_store(c, slot)

                @pl.when(c + _NBUF < NCH)
                def _():
                    start_gather(c + _NBUF, slot)

            return carry

        lax.fori_loop(0, NCH // _NBUF, pair_body, 0)

    return k(x_pad, table)


def _tc_add_pos(raw, pos_pad, B, T):
    _, TP, D = raw.shape

    def body(raw_ref, pos_ref, out_ref):
        out_ref[0] = raw_ref[0, :T, :] + pos_ref[:T, :]

    return pl.pallas_call(
        body,
        grid=(B,),
        in_specs=[
            pl.BlockSpec((1, TP, D), lambda b: (b, 0, 0)),
            pl.BlockSpec((TP, D), lambda b: (0, 0)),
        ],
        out_specs=pl.BlockSpec((1, T, D), lambda b: (b, 0, 0)),
        out_shape=jax.ShapeDtypeStruct((B, T, D), jnp.float32),
    )(raw, pos_pad)


def kernel(x, token_embedding, position_embedding):
    B, T = x.shape
    TP = ((T + _LANES - 1) // _LANES) * _LANES
    x_pad = jnp.pad(x.astype(jnp.int32), ((0, 0), (0, TP - T))).reshape(-1)
    pos_pad = jnp.pad(position_embedding, ((0, TP - T), (0, 0)))
    raw = _sc_gather(x_pad, token_embedding, B, TP)
    return _tc_add_pos(raw, pos_pad, B, T)


# speedup vs baseline: 1.5849x; 1.5849x over previous
import functools

import jax
import jax.numpy as jnp
from jax import lax
from jax.experimental import pallas as pl
from jax.experimental.pallas import tpu as pltpu
from jax.experimental.pallas import tpu_sc as plsc

_NUM_CORES = 2
_NUM_SUBCORES = 16
_LANES = 16
_NUM_WORKERS = _NUM_CORES * _NUM_SUBCORES
_CHUNK = 40
_NBUF = 2


def _sc_gather(x_pad, table, B, TP):
    (N,) = x_pad.shape
    V, D = table.shape
    BW = B // _NUM_WORKERS
    RP = BW * TP
    CPB = TP // _CHUNK
    NCH = BW * CPB

    mesh = plsc.VectorSubcoreMesh(core_axis_name="c", subcore_axis_name="s")

    @functools.partial(
        pl.kernel,
        out_type=jax.ShapeDtypeStruct((B, TP, D), jnp.float32),
        mesh=mesh,
        compiler_params=pltpu.CompilerParams(use_tc_tiling_on_sc=True),
        scratch_types=[
            pltpu.VMEM((RP,), jnp.int32),
            pltpu.VMEM((_CHUNK, D), jnp.float32),
            pltpu.VMEM((_CHUNK, D), jnp.float32),
            pltpu.SemaphoreType.DMA,
            pltpu.SemaphoreType.DMA,
            pltpu.SemaphoreType.DMA,
            pltpu.SemaphoreType.DMA,
        ],
    )
    def k(x_hbm, tab_hbm, out_hbm, idx_v, lin0, lin1, g0, g1, s0, s1):
        wid = lax.axis_index("s") * _NUM_CORES + lax.axis_index("c")
        b0 = wid * BW
        lins = [lin0, lin1]
        gsems = [g0, g1]
        ssems = [s0, s1]

        pltpu.sync_copy(x_hbm.at[pl.ds(wid * RP, RP)], idx_v)

        def start_gather(c, slot):
            pltpu.async_copy(
                tab_hbm.at[idx_v.at[pl.ds(c * _CHUNK, _CHUNK)]],
                lins[slot], gsems[slot],
            )

        def wait_gather(slot):
            pltpu.make_async_copy(
                tab_hbm.at[idx_v.at[pl.ds(0, _CHUNK)]],
                lins[slot], gsems[slot],
            ).wait()

        def out_ref(c):
            bb = c // CPB
            piece = lax.rem(c, CPB)
            return out_hbm.at[b0 + bb, pl.ds(piece * _CHUNK, _CHUNK)]

        def start_store(c, slot):
            pltpu.async_copy(lins[slot], out_ref(c), ssems[slot])

        def wait_store(c, slot):
            pltpu.make_async_copy(lins[slot], out_ref(c), ssems[slot]).wait()

        for slot in range(_NBUF):
            start_gather(slot, slot)

        def pair_body(p, carry):
            for slot in range(_NBUF):
                c = p * _NBUF + slot
                wait_gather(slot)
                start_store(c, slot)
                wait_store(c, slot)

                @pl.when(c + _NBUF < NCH)
                def _():
                    start_gather(c + _NBUF, slot)

            return carry

        lax.fori_loop(0, NCH // _NBUF, pair_body, 0)

    return k(x_pad, table)


def _tc_add_pos(raw, pos_pad, B, T):
    _, TP, D = raw.shape
    BB = 8

    def body(raw_ref, pos_ref, out_ref):
        p = pos_ref[:T, :]
        out_ref[...] = raw_ref[:, :T, :] + p[None, :, :]

    return pl.pallas_call(
        body,
        grid=(B // BB,),
        in_specs=[
            pl.BlockSpec((BB, TP, D), lambda b: (b, 0, 0)),
            pl.BlockSpec((TP, D), lambda b: (0, 0)),
        ],
        out_specs=pl.BlockSpec((BB, T, D), lambda b: (b, 0, 0)),
        out_shape=jax.ShapeDtypeStruct((B, T, D), jnp.float32),
    )(raw, pos_pad)


def kernel(x, token_embedding, position_embedding):
    B, T = x.shape
    TP = ((T + _LANES - 1) // _LANES) * _LANES
    x_pad = jnp.pad(x.astype(jnp.int32), ((0, 0), (0, TP - T))).reshape(-1)
    pos_pad = jnp.pad(position_embedding, ((0, TP - T), (0, 0)))
    raw = _sc_gather(x_pad, token_embedding, B, TP)
    return _tc_add_pos(raw, pos_pad, B, T)


# speedup vs baseline: 1.6156x; 1.0194x over previous
import functools

import jax
import jax.numpy as jnp
from jax import lax
from jax.experimental import pallas as pl
from jax.experimental.pallas import tpu as pltpu
from jax.experimental.pallas import tpu_sc as plsc

_NUM_CORES = 2
_NUM_SUBCORES = 16
_LANES = 16
_NUM_WORKERS = _NUM_CORES * _NUM_SUBCORES
_CHUNK = 40
_NBUF = 2


def _sc_gather(x_pad, table, B, TP):
    (N,) = x_pad.shape
    V, D = table.shape
    BW = B // _NUM_WORKERS
    RP = BW * TP
    CPB = TP // _CHUNK
    NCH = BW * CPB

    mesh = plsc.VectorSubcoreMesh(core_axis_name="c", subcore_axis_name="s")

    @functools.partial(
        pl.kernel,
        out_type=jax.ShapeDtypeStruct((B, TP, D), jnp.float32),
        mesh=mesh,
        compiler_params=pltpu.CompilerParams(use_tc_tiling_on_sc=True),
        scratch_types=[
            pltpu.VMEM((RP,), jnp.int32),
            pltpu.VMEM((_CHUNK, D), jnp.float32),
            pltpu.VMEM((_CHUNK, D), jnp.float32),
            pltpu.SemaphoreType.DMA,
            pltpu.SemaphoreType.DMA,
            pltpu.SemaphoreType.DMA,
            pltpu.SemaphoreType.DMA,
        ],
    )
    def k(x_hbm, tab_hbm, out_hbm, idx_v, lin0, lin1, g0, g1, s0, s1):
        wid = lax.axis_index("s") * _NUM_CORES + lax.axis_index("c")
        b0 = wid * BW
        lins = [lin0, lin1]
        gsems = [g0, g1]
        ssems = [s0, s1]

        pltpu.sync_copy(x_hbm.at[pl.ds(wid * RP, RP)], idx_v)

        def start_gather(c, slot):
            pltpu.async_copy(
                tab_hbm.at[idx_v.at[pl.ds(c * _CHUNK, _CHUNK)]],
                lins[slot], gsems[slot],
            )

        def wait_gather(slot):
            pltpu.make_async_copy(
                tab_hbm.at[idx_v.at[pl.ds(0, _CHUNK)]],
                lins[slot], gsems[slot],
            ).wait()

        def out_ref(c):
            bb = c // CPB
            piece = lax.rem(c, CPB)
            return out_hbm.at[b0 + bb, pl.ds(piece * _CHUNK, _CHUNK)]

        def start_store(c, slot):
            pltpu.async_copy(lins[slot], out_ref(c), ssems[slot])

        def wait_store(c, slot):
            pltpu.make_async_copy(lins[slot], out_ref(c), ssems[slot]).wait()

        for slot in range(_NBUF):
            start_gather(slot, slot)

        def pair_body(p, carry):
            for slot in range(_NBUF):
                c = p * _NBUF + slot
                wait_gather(slot)
                start_store(c, slot)
                wait_store(c, slot)

                @pl.when(c + _NBUF < NCH)
                def _():
                    start_gather(c + _NBUF, slot)

            return carry

        lax.fori_loop(0, NCH // _NBUF, pair_body, 0)

    return k(x_pad, table)


def _tc_add_pos(raw, pos_pad, B, T):
    _, TP, D = raw.shape
    BB = 16

    def body(raw_ref, pos_ref, out_ref):
        p = pos_ref[:T, :]
        out_ref[...] = raw_ref[:, :T, :] + p[None, :, :]

    return pl.pallas_call(
        body,
        grid=(B // BB,),
        in_specs=[
            pl.BlockSpec((BB, TP, D), lambda b: (b, 0, 0)),
            pl.BlockSpec((TP, D), lambda b: (0, 0)),
        ],
        out_specs=pl.BlockSpec((BB, T, D), lambda b: (b, 0, 0)),
        out_shape=jax.ShapeDtypeStruct((B, T, D), jnp.float32),
    )(raw, pos_pad)


def kernel(x, token_embedding, position_embedding):
    B, T = x.shape
    TP = ((T + _LANES - 1) // _LANES) * _LANES
    x_pad = jnp.pad(x.astype(jnp.int32), ((0, 0), (0, TP - T))).reshape(-1)
    pos_pad = jnp.pad(position_embedding, ((0, TP - T), (0, 0)))
    raw = _sc_gather(x_pad, token_embedding, B, TP)
    return _tc_add_pos(raw, pos_pad, B, T)
